# baseline (speedup 1.0000x reference)
import functools

import jax
import jax.numpy as jnp
from jax import lax
from jax.experimental import pallas as pl
from jax.experimental.pallas import tpu as pltpu
from jax.experimental.pallas import tpu_sc as plsc

N_NODES = 10000
N_PAD = 10240
N_EDGES = 320000
NC, NS = 2, 16
NW = NC * NS
EW = N_EDGES // NW
K = 80
NCHUNK = EW // K
RPS = N_PAD // NS
NB = 5

_mesh = plsc.VectorSubcoreMesh(
    core_axis_name="c", subcore_axis_name="s", num_cores=NC, num_subcores=NS)


@functools.partial(
    pl.kernel,
    out_type=jax.ShapeDtypeStruct((NC, N_PAD), jnp.float32),
    mesh=_mesh,
    scratch_types=[
        pltpu.VMEM_SHARED((N_PAD,), jnp.float32),
        pltpu.VMEM((EW,), jnp.int32),
        pltpu.VMEM((K,), jnp.float32),
        pltpu.VMEM((RPS,), jnp.float32),
        pltpu.SemaphoreType.DMA,
    ],
)
def _deg_kernel(dst_hbm, out_hbm, deg_sp, dst_v, ones_v, zb, sem):
    cid = lax.axis_index("c")
    sid = lax.axis_index("s")
    wid = cid * NS + sid

    @pl.loop(0, RPS // 16)
    def _zero(i):
        zb[pl.ds(i * 16, 16)] = jnp.zeros((16,), jnp.float32)

    for i in range(K // 16):
        ones_v[pl.ds(i * 16, 16)] = jnp.ones((16,), jnp.float32)

    pltpu.sync_copy(zb, deg_sp.at[pl.ds(sid * RPS, RPS)])
    plsc.subcore_barrier()

    pltpu.sync_copy(dst_hbm.at[pl.ds(wid * EW, EW)], dst_v)

    @pl.loop(0, NCHUNK, step=NB * NB)
    def _scat(j):
        ds = [pltpu.async_copy(ones_v,
                               deg_sp.at[dst_v.at[pl.ds((j + b) * K, K)]],
                               sem, add=True) for b in range(NB * NB)]
        for d in ds:
            d.wait()

    plsc.subcore_barrier()
    pltpu.sync_copy(deg_sp.at[pl.ds(sid * RPS, RPS)],
                    out_hbm.at[cid, pl.ds(sid * RPS, RPS)])


def _make_edge_scatter(dim):
    @functools.partial(
        pl.kernel,
        out_type=jax.ShapeDtypeStruct((NC, N_PAD, dim), jnp.float32),
        mesh=_mesh,
        scratch_types=(
            [
                pltpu.VMEM_SHARED((N_PAD, dim), jnp.float32),
                pltpu.VMEM((EW,), jnp.int32),
                pltpu.VMEM((EW,), jnp.int32),
                pltpu.VMEM((64, dim), jnp.float32),
            ]
            + [pltpu.VMEM((K, dim), jnp.float32) for _ in range(2 * NB)]
            + [pltpu.SemaphoreType.DMA for _ in range(4 * NB)]
        ),
        compiler_params=pltpu.CompilerParams(use_tc_tiling_on_sc=False),
    )
    def _edge_scatter(hp_hbm, src_hbm, dst_hbm, out_hbm,
                      acc, src_v, dst_v, zb, *bufs):
        rows, sems = bufs[:2 * NB], bufs[2 * NB:]
        rA, rB = rows[:NB], rows[NB:]
        gA, gB = sems[0:NB], sems[NB:2 * NB]
        sA, sB = sems[2 * NB:3 * NB], sems[3 * NB:]
        cid = lax.axis_index("c")
        sid = lax.axis_index("s")
        wid = cid * NS + sid

        @pl.loop(0, 64)
        def _zero(i):
            for c in range(dim // 16):
                zb[i, pl.ds(c * 16, 16)] = jnp.zeros((16,), jnp.float32)

        @pl.loop(0, RPS // 64)
        def _clr(t):
            pltpu.sync_copy(zb, acc.at[pl.ds(sid * RPS + t * 64, 64)])

        plsc.subcore_barrier()

        pltpu.sync_copy(src_hbm.at[pl.ds(wid * EW, EW)], src_v)
        pltpu.sync_copy(dst_hbm.at[pl.ds(wid * EW, EW)], dst_v)

        def _issue_g(base, r, g):
            for b in range(NB):
                pltpu.async_copy(
                    hp_hbm.at[src_v.at[pl.ds((base + b) * K, K)]], r[b], g[b])

        def _wait_g(base, r, g):
            for b in range(NB):
                pltpu.make_async_copy(
                    hp_hbm.at[src_v.at[pl.ds((base + b) * K, K)]],
                    r[b], g[b]).wait()

        def _issue_s(base, r, s):
            return [pltpu.async_copy(
                r[b], acc.at[dst_v.at[pl.ds((base + b) * K, K)]],
                s[b], add=True) for b in range(NB)]

        ngroups = NCHUNK // NB
        _issue_g(0, rA, gA)

        @pl.loop(0, ngroups - 1, step=2)
        def _edges(g):
            j = g * NB
            _wait_g(j, rA, gA)
            dsA = _issue_s(j, rA, sA)
            _issue_g(j + NB, rB, gB)
            for d in dsA:
                d.wait()
            _wait_g(j + NB, rB, gB)
            dsB = _issue_s(j + NB, rB, sB)
            _issue_g(j + 2 * NB, rA, gA)
            for d in dsB:
                d.wait()

        jl = (ngroups - 1) * NB
        _wait_g(jl, rA, gA)
        for d in _issue_s(jl, rA, sA):
            d.wait()

        plsc.subcore_barrier()
        pltpu.sync_copy(acc.at[pl.ds(sid * RPS, RPS)],
                        out_hbm.at[cid, pl.ds(sid * RPS, RPS)])

    return _edge_scatter


_edge_scatter64 = _make_edge_scatter(64)


_BM = 1024
_GRID = N_PAD // _BM


def _mm1_body(degp_ref, x_ref, w_ref, hp_ref, dinv_ref):
    deg = degp_ref[0, :] + degp_ref[1, :] + 1.0
    dinv = lax.rsqrt(deg)[:, None]
    h = jnp.dot(x_ref[...], w_ref[...], preferred_element_type=jnp.float32)
    hp_ref[...] = h * dinv
    dinv_ref[...] = dinv


def _mm2_body(accp_ref, hp1_ref, dinv_ref, b1_ref, w2_ref, hp2_ref):
    dinv = dinv_ref[...]
    tot = accp_ref[0] + accp_ref[1] + hp1_ref[...]
    h2 = jnp.maximum(tot * dinv + b1_ref[...], 0.0)
    h = jnp.dot(h2, w2_ref[...], preferred_element_type=jnp.float32)
    hp2_ref[...] = h * dinv


def _out_body(accp_ref, hp2_ref, dinv_ref, b2_ref, out_ref):
    tot = accp_ref[0] + accp_ref[1] + hp2_ref[...]
    out_ref[...] = tot * dinv_ref[...] + b2_ref[...]


def _mm1(deg_parts, x, W1):
    return pl.pallas_call(
        _mm1_body,
        grid=(_GRID,),
        in_specs=[
            pl.BlockSpec((NC, _BM), lambda i: (0, i)),
            pl.BlockSpec((_BM, 128), lambda i: (i, 0)),
            pl.BlockSpec((128, 64), lambda i: (0, 0)),
        ],
        out_specs=[
            pl.BlockSpec((_BM, 64), lambda i: (i, 0)),
            pl.BlockSpec((_BM, 1), lambda i: (i, 0)),
        ],
        out_shape=[
            jax.ShapeDtypeStruct((N_PAD, 64), jnp.float32),
            jax.ShapeDtypeStruct((N_PAD, 1), jnp.float32),
        ],
    )(deg_parts, x, W1)


def _mm2(acc_parts, hp1, dinv, b1, W2):
    return pl.pallas_call(
        _mm2_body,
        grid=(_GRID,),
        in_specs=[
            pl.BlockSpec((NC, _BM, 64), lambda i: (0, i, 0)),
            pl.BlockSpec((_BM, 64), lambda i: (i, 0)),
            pl.BlockSpec((_BM, 1), lambda i: (i, 0)),
            pl.BlockSpec((1, 64), lambda i: (0, 0)),
            pl.BlockSpec((64, 64), lambda i: (0, 0)),
        ],
        out_specs=pl.BlockSpec((_BM, 64), lambda i: (i, 0)),
        out_shape=jax.ShapeDtypeStruct((N_PAD, 64), jnp.float32),
    )(acc_parts, hp1, dinv, b1, W2)


_BMO = 1000


def _out(acc_parts, hp2, dinv, b2):
    return pl.pallas_call(
        _out_body,
        grid=(N_NODES // _BMO,),
        in_specs=[
            pl.BlockSpec((NC, _BMO, 64), lambda i: (0, i, 0)),
            pl.BlockSpec((_BMO, 64), lambda i: (i, 0)),
            pl.BlockSpec((_BMO, 1), lambda i: (i, 0)),
            pl.BlockSpec((1, 64), lambda i: (0, 0)),
        ],
        out_specs=pl.BlockSpec((_BMO, 64), lambda i: (i, 0)),
        out_shape=jax.ShapeDtypeStruct((N_NODES, 64), jnp.float32),
    )(acc_parts, hp2, dinv, b2)


def kernel(x, edge_index, W1, b1, W2, b2):
    src = edge_index[0].astype(jnp.int32)
    dst = edge_index[1].astype(jnp.int32)
    x_p = jnp.pad(x, ((0, N_PAD - N_NODES), (0, 0)))
    b1r = b1.reshape(1, 64)
    b2r = b2.reshape(1, 64)

    deg_parts = _deg_kernel(dst)
    hp1, dinv = _mm1(deg_parts, x_p, W1)
    acc1 = _edge_scatter64(hp1, src, dst)
    hp2 = _mm2(acc1, hp1, dinv, b1r, W2)
    acc2 = _edge_scatter64(hp2, src, dst)
    return _out(acc2, hp2, dinv, b2r)


# speedup vs baseline: 1.0284x; 1.0284x over previous
import functools

import jax
import jax.numpy as jnp
from jax import lax
from jax.experimental import pallas as pl
from jax.experimental.pallas import tpu as pltpu
from jax.experimental.pallas import tpu_sc as plsc

N_NODES = 10000
N_PAD = 10240
N_EDGES = 320000
NC, NS = 2, 16
NW = NC * NS
EW = N_EDGES // NW
K = 80
NCHUNK = EW // K
RPS = N_PAD // NS
NB = 5

_mesh = plsc.VectorSubcoreMesh(
    core_axis_name="c", subcore_axis_name="s", num_cores=NC, num_subcores=NS)


@functools.partial(
    pl.kernel,
    out_type=jax.ShapeDtypeStruct((NC, N_PAD), jnp.float32),
    mesh=_mesh,
    scratch_types=[
        pltpu.VMEM_SHARED((N_PAD,), jnp.float32),
        pltpu.VMEM((EW,), jnp.int32),
        pltpu.VMEM((K,), jnp.float32),
        pltpu.VMEM((RPS,), jnp.float32),
        pltpu.SemaphoreType.DMA,
    ],
)
def _deg_kernel(dst_hbm, out_hbm, deg_sp, dst_v, ones_v, zb, sem):
    cid = lax.axis_index("c")
    sid = lax.axis_index("s")
    wid = cid * NS + sid

    @pl.loop(0, RPS // 16)
    def _zero(i):
        zb[pl.ds(i * 16, 16)] = jnp.zeros((16,), jnp.float32)

    for i in range(K // 16):
        ones_v[pl.ds(i * 16, 16)] = jnp.ones((16,), jnp.float32)

    pltpu.sync_copy(zb, deg_sp.at[pl.ds(sid * RPS, RPS)])
    plsc.subcore_barrier()

    pltpu.sync_copy(dst_hbm.at[pl.ds(wid * EW, EW)], dst_v)

    @pl.loop(0, NCHUNK, step=NB * NB)
    def _scat(j):
        ds = [pltpu.async_copy(ones_v,
                               deg_sp.at[dst_v.at[pl.ds((j + b) * K, K)]],
                               sem, add=True) for b in range(NB * NB)]
        for d in ds:
            d.wait()

    plsc.subcore_barrier()
    pltpu.sync_copy(deg_sp.at[pl.ds(sid * RPS, RPS)],
                    out_hbm.at[cid, pl.ds(sid * RPS, RPS)])


def _make_edge_scatter(dim):
    @functools.partial(
        pl.kernel,
        out_type=jax.ShapeDtypeStruct((NC, N_PAD, dim), jnp.float32),
        mesh=_mesh,
        scratch_types=(
            [
                pltpu.VMEM_SHARED((N_PAD, dim), jnp.float32),
                pltpu.VMEM((EW,), jnp.int32),
                pltpu.VMEM((EW,), jnp.int32),
                pltpu.VMEM((64, dim), jnp.float32),
            ]
            + [pltpu.VMEM((K, dim), jnp.float32) for _ in range(2 * NB)]
            + [pltpu.SemaphoreType.DMA for _ in range(4 * NB)]
        ),
        compiler_params=pltpu.CompilerParams(use_tc_tiling_on_sc=False),
    )
    def _edge_scatter(hp_hbm, src_hbm, dst_hbm, out_hbm,
                      acc, src_v, dst_v, zb, *bufs):
        rows, sems = bufs[:2 * NB], bufs[2 * NB:]
        rA, rB = rows[:NB], rows[NB:]
        gA, gB = sems[0:NB], sems[NB:2 * NB]
        sA, sB = sems[2 * NB:3 * NB], sems[3 * NB:]
        cid = lax.axis_index("c")
        sid = lax.axis_index("s")
        wid = cid * NS + sid

        @pl.loop(0, 64)
        def _zero(i):
            for c in range(dim // 16):
                zb[i, pl.ds(c * 16, 16)] = jnp.zeros((16,), jnp.float32)

        @pl.loop(0, RPS // 64)
        def _clr(t):
            pltpu.sync_copy(zb, acc.at[pl.ds(sid * RPS + t * 64, 64)])

        plsc.subcore_barrier()

        pltpu.sync_copy(src_hbm.at[pl.ds(wid * EW, EW)], src_v)
        pltpu.sync_copy(dst_hbm.at[pl.ds(wid * EW, EW)], dst_v)

        def _issue_g(base, r, g):
            for b in range(NB):
                pltpu.async_copy(
                    hp_hbm.at[src_v.at[pl.ds((base + b) * K, K)]], r[b], g[b])

        def _wait_g(base, r, g):
            for b in range(NB):
                pltpu.make_async_copy(
                    hp_hbm.at[src_v.at[pl.ds((base + b) * K, K)]],
                    r[b], g[b]).wait()

        def _issue_s(base, r, s):
            return [pltpu.async_copy(
                r[b], acc.at[dst_v.at[pl.ds((base + b) * K, K)]],
                s[b], add=True) for b in range(NB)]

        ngroups = NCHUNK // NB
        _issue_g(0, rA, gA)

        @pl.loop(0, ngroups - 1, step=2)
        def _edges(g):
            j = g * NB
            _wait_g(j, rA, gA)
            dsA = _issue_s(j, rA, sA)
            _issue_g(j + NB, rB, gB)
            for d in dsA:
                d.wait()
            _wait_g(j + NB, rB, gB)
            dsB = _issue_s(j + NB, rB, sB)
            _issue_g(j + 2 * NB, rA, gA)
            for d in dsB:
                d.wait()

        jl = (ngroups - 1) * NB
        _wait_g(jl, rA, gA)
        for d in _issue_s(jl, rA, sA):
            d.wait()

        plsc.subcore_barrier()
        pltpu.sync_copy(acc.at[pl.ds(sid * RPS, RPS)],
                        out_hbm.at[cid, pl.ds(sid * RPS, RPS)])

    return _edge_scatter


_edge_scatter64 = _make_edge_scatter(64)


def _mm1_body(degp_ref, x_ref, w_ref, hp_ref, dinv_ref):
    deg = degp_ref[0, :] + degp_ref[1, :] + 1.0
    dinv = lax.rsqrt(deg)[:, None]
    h = jnp.dot(x_ref[...], w_ref[...], preferred_element_type=jnp.float32)
    hp_ref[...] = h * dinv
    dinv_ref[...] = dinv


def _mm2_body(accp_ref, hp1_ref, dinv_ref, b1_ref, w2_ref, hp2_ref):
    dinv = dinv_ref[...]
    tot = accp_ref[0] + accp_ref[1] + hp1_ref[...]
    h2 = jnp.maximum(tot * dinv + b1_ref[...], 0.0)
    h = jnp.dot(h2, w2_ref[...], preferred_element_type=jnp.float32)
    hp2_ref[...] = h * dinv


def _out_body(accp_ref, hp2_ref, dinv_ref, b2_ref, out_ref):
    tot = (accp_ref[0, :N_NODES] + accp_ref[1, :N_NODES]
           + hp2_ref[:N_NODES])
    out_ref[...] = tot * dinv_ref[:N_NODES] + b2_ref[...]


def _mm1(deg_parts, x, W1):
    return pl.pallas_call(
        _mm1_body,
        out_shape=[
            jax.ShapeDtypeStruct((N_PAD, 64), jnp.float32),
            jax.ShapeDtypeStruct((N_PAD, 1), jnp.float32),
        ],
    )(deg_parts, x, W1)


def _mm2(acc_parts, hp1, dinv, b1, W2):
    return pl.pallas_call(
        _mm2_body,
        out_shape=jax.ShapeDtypeStruct((N_PAD, 64), jnp.float32),
    )(acc_parts, hp1, dinv, b1, W2)


def _out(acc_parts, hp2, dinv, b2):
    return pl.pallas_call(
        _out_body,
        out_shape=jax.ShapeDtypeStruct((N_NODES, 64), jnp.float32),
    )(acc_parts, hp2, dinv, b2)


def kernel(x, edge_index, W1, b1, W2, b2):
    src = edge_index[0].astype(jnp.int32)
    dst = edge_index[1].astype(jnp.int32)
    x_p = jnp.pad(x, ((0, N_PAD - N_NODES), (0, 0)))
    b1r = b1.reshape(1, 64)
    b2r = b2.reshape(1, 64)

    deg_parts = _deg_kernel(dst)
    hp1, dinv = _mm1(deg_parts, x_p, W1)
    acc1 = _edge_scatter64(hp1, src, dst)
    hp2 = _mm2(acc1, hp1, dinv, b1r, W2)
    acc2 = _edge_scatter64(hp2, src, dst)
    return _out(acc2, hp2, dinv, b2r)
